# Initial kernel scaffold; baseline (speedup 1.0000x reference)
import functools

import jax
import jax.numpy as jnp
from jax import lax
from jax.experimental import pallas as pl
from jax.experimental.pallas import tpu as pltpu
from jax.experimental.pallas import tpu_sc as plsc

T = 2048
H = 768
F = 3072
E = 8
BT = 128
NB = 40
NPAD = NB * BT

NC, NS = 2, 16
NW = NC * NS
TPW = T // NW


def _router_body(x_ref, rw_ref, pos1_ref, pos2_ref, w1_ref, w2_ref,
                 bexp_ref, bval_ref):
    xv = x_ref[...]
    logits = jnp.dot(xv, rw_ref[...], preferred_element_type=jnp.float32)
    ie = lax.broadcasted_iota(jnp.int32, (T, E), 1)
    m1 = jnp.max(logits, axis=1, keepdims=True)
    e1 = jnp.min(jnp.where(logits == m1, ie, E), axis=1, keepdims=True)
    masked = jnp.where(ie == e1, -jnp.inf, logits)
    m2 = jnp.max(masked, axis=1, keepdims=True)
    e2 = jnp.min(jnp.where(masked == m2, ie, E), axis=1, keepdims=True)
    w1 = jax.nn.sigmoid(m1 - m2)
    w1_ref[...] = w1
    w2_ref[...] = 1.0 - w1

    oh1 = (ie == e1).astype(jnp.float32)
    oh2 = (ie == e2).astype(jnp.float32)
    ohb = jnp.concatenate([oh1, oh2], axis=1)
    it = lax.broadcasted_iota(jnp.int32, (T, T), 0)
    jt = lax.broadcasted_iota(jnp.int32, (T, T), 1)
    tri = (jt < it).astype(jnp.float32)
    cb = jnp.dot(tri, ohb, preferred_element_type=jnp.float32)
    c1 = cb[:, :E]
    c2 = cb[:, E:]
    cnt1 = jnp.sum(oh1, axis=0, keepdims=True)
    cnt2 = jnp.sum(oh2, axis=0, keepdims=True)
    cnt = cnt1 + cnt2
    used = jnp.floor((cnt + (BT - 1)) * (1.0 / BT))

    iee = lax.broadcasted_iota(jnp.int32, (E, E), 0)
    jee = lax.broadcasted_iota(jnp.int32, (E, E), 1)
    upper = (iee < jee).astype(jnp.float32)
    used8 = jnp.broadcast_to(used, (E, E))
    start = jnp.dot(used8, upper, preferred_element_type=jnp.float32)[0:1]
    pad_off = start * BT

    pos1 = jnp.sum(oh1 * (pad_off + c1), axis=1, keepdims=True)
    pos2 = jnp.sum(oh2 * (pad_off + cnt1 + c2), axis=1, keepdims=True)
    pos1_ref[...] = pos1.astype(jnp.int32)
    pos2_ref[...] = pos2.astype(jnp.int32)

    ibf = lax.broadcasted_iota(jnp.float32, (NB, E), 0)
    ebf = lax.broadcasted_iota(jnp.float32, (NB, E), 1)
    startb = jnp.broadcast_to(start, (NB, E))
    usedb = jnp.broadcast_to(used, (NB, E))
    inr = jnp.logical_and(ibf >= startb, ibf < startb + usedb)
    bexp = jnp.sum(jnp.where(inr, ebf, 0.0), axis=1, keepdims=True)
    bval = jnp.sum(jnp.where(inr, 1.0, 0.0), axis=1, keepdims=True)
    ef = lax.broadcasted_iota(jnp.float32, (1, E), 1)
    laste = jnp.max(jnp.where(used > 0, ef, -1.0))
    bexp_ref[...] = jnp.where(bval > 0, bexp, laste).astype(jnp.int32)
    bval_ref[...] = (bval > 0).astype(jnp.int32)


_router = pl.pallas_call(
    _router_body,
    out_shape=(
        jax.ShapeDtypeStruct((T, 1), jnp.int32),
        jax.ShapeDtypeStruct((T, 1), jnp.int32),
        jax.ShapeDtypeStruct((T, 1), jnp.float32),
        jax.ShapeDtypeStruct((T, 1), jnp.float32),
        jax.ShapeDtypeStruct((NB, 1), jnp.int32),
        jax.ShapeDtypeStruct((NB, 1), jnp.int32),
    ),
)


@functools.partial(
    pl.kernel,
    out_type=(
        jax.ShapeDtypeStruct((NPAD, H), jnp.float32),
        jax.ShapeDtypeStruct((NPAD,), jnp.float32),
    ),
    mesh=plsc.VectorSubcoreMesh(core_axis_name="c", subcore_axis_name="s"),
    scratch_types=[
        pltpu.VMEM((TPW, H), jnp.float32),
        pltpu.VMEM((TPW,), jnp.int32),
        pltpu.VMEM((TPW,), jnp.int32),
        pltpu.VMEM((TPW,), jnp.float32),
        pltpu.VMEM((TPW,), jnp.float32),
        pltpu.SemaphoreType.DMA,
        pltpu.SemaphoreType.DMA,
        pltpu.SemaphoreType.DMA,
        pltpu.SemaphoreType.DMA,
    ],
)
def _dispatch(x_hbm, pos1_hbm, pos2_hbm, w1_hbm, w2_hbm, xs_hbm, wpad_hbm,
              xrows, p1v, p2v, w1v, w2v, s1, s2, s3, s4):
    wid = lax.axis_index("s") * NC + lax.axis_index("c")
    base = wid * TPW
    pltpu.sync_copy(x_hbm.at[pl.ds(base, TPW)], xrows)
    pltpu.sync_copy(pos1_hbm.at[pl.ds(base, TPW)], p1v)
    pltpu.sync_copy(pos2_hbm.at[pl.ds(base, TPW)], p2v)
    pltpu.sync_copy(w1_hbm.at[pl.ds(base, TPW)], w1v)
    pltpu.sync_copy(w2_hbm.at[pl.ds(base, TPW)], w2v)
    c1 = pltpu.async_copy(xrows, xs_hbm.at[p1v], s1)
    c2 = pltpu.async_copy(xrows, xs_hbm.at[p2v], s2)
    c3 = pltpu.async_copy(w1v, wpad_hbm.at[p1v], s3)
    c4 = pltpu.async_copy(w2v, wpad_hbm.at[p2v], s4)
    c1.wait()
    c2.wait()
    c3.wait()
    c4.wait()


def _ffn_body(bexp_ref, bval_ref, xs_ref, wpad_ref, W1_ref, W3_ref, W2_ref,
              ys_ref):
    b = pl.program_id(0)

    @pl.when(bval_ref[b] != 0)
    def _():
        xb = xs_ref[...]
        h1 = jnp.dot(xb, W1_ref[0], preferred_element_type=jnp.float32)
        h3 = jnp.dot(xb, W3_ref[0], preferred_element_type=jnp.float32)
        act = h1 * jax.nn.sigmoid(h1) * h3
        y = jnp.dot(act, W2_ref[0], preferred_element_type=jnp.float32)
        ys_ref[...] = y * wpad_ref[...]


_ffn = pl.pallas_call(
    _ffn_body,
    grid_spec=pltpu.PrefetchScalarGridSpec(
        num_scalar_prefetch=2,
        grid=(NB,),
        in_specs=[
            pl.BlockSpec((BT, H), lambda b, be, bv: (b, 0)),
            pl.BlockSpec((BT, 1), lambda b, be, bv: (b, 0)),
            pl.BlockSpec((1, H, F), lambda b, be, bv: (be[b], 0, 0)),
            pl.BlockSpec((1, H, F), lambda b, be, bv: (be[b], 0, 0)),
            pl.BlockSpec((1, F, H), lambda b, be, bv: (be[b], 0, 0)),
        ],
        out_specs=pl.BlockSpec((BT, H), lambda b, be, bv: (b, 0)),
    ),
    out_shape=jax.ShapeDtypeStruct((NPAD, H), jnp.float32),
)


@functools.partial(
    pl.kernel,
    out_type=jax.ShapeDtypeStruct((T, H), jnp.float32),
    mesh=plsc.VectorSubcoreMesh(core_axis_name="c", subcore_axis_name="s"),
    scratch_types=[
        pltpu.VMEM((TPW, H), jnp.float32),
        pltpu.VMEM((TPW, H), jnp.float32),
        pltpu.VMEM((TPW,), jnp.int32),
        pltpu.VMEM((TPW,), jnp.int32),
        pltpu.SemaphoreType.DMA,
        pltpu.SemaphoreType.DMA,
    ],
)
def _combine(ys_hbm, pos1_hbm, pos2_hbm, out_hbm, y1v, y2v, p1v, p2v, s1, s2):
    wid = lax.axis_index("s") * NC + lax.axis_index("c")
    base = wid * TPW
    pltpu.sync_copy(pos1_hbm.at[pl.ds(base, TPW)], p1v)
    pltpu.sync_copy(pos2_hbm.at[pl.ds(base, TPW)], p2v)
    c1 = pltpu.async_copy(ys_hbm.at[p1v], y1v, s1)
    c2 = pltpu.async_copy(ys_hbm.at[p2v], y2v, s2)
    c1.wait()
    c2.wait()

    def row(r, carry):
        for c0 in range(0, H, 16):
            y1v[r, pl.ds(c0, 16)] = y1v[r, pl.ds(c0, 16)] + y2v[r, pl.ds(c0, 16)]
        return carry

    lax.fori_loop(0, TPW, row, 0)
    pltpu.sync_copy(y1v, out_hbm.at[pl.ds(base, TPW)])


def kernel(x, router_w, W1, W3, W2):
    b, l, h = x.shape
    x2 = x.reshape(T, H)
    pos1, pos2, w1, w2, bexp, bval = _router(x2, router_w)
    pos1 = pos1.reshape(T)
    pos2 = pos2.reshape(T)
    xs, wpad = _dispatch(x2, pos1, pos2, w1.reshape(T), w2.reshape(T))
    ys = _ffn(bexp.reshape(NB), bval.reshape(NB), xs, wpad.reshape(NPAD, 1),
              W1, W3, W2)
    out = _combine(ys, pos1, pos2)
    return out.reshape(b, l, h)


# speedup vs baseline: 1.5424x; 1.5424x over previous
import functools

import jax
import jax.numpy as jnp
from jax import lax
from jax.experimental import pallas as pl
from jax.experimental.pallas import tpu as pltpu
from jax.experimental.pallas import tpu_sc as plsc

T = 2048
H = 768
F = 3072
E = 8
BT = 256
NB = 24
NPAD = NB * BT
FC = 768
NFC = F // FC

NC, NS = 2, 16
NW = NC * NS
TPW = T // NW


def _router_body(x_ref, rw_ref, pos1_ref, pos2_ref, w1_ref, w2_ref,
                 bexp_ref, bval_ref):
    xv = x_ref[...]
    logits = jnp.dot(xv, rw_ref[...], preferred_element_type=jnp.float32)
    ie = lax.broadcasted_iota(jnp.int32, (T, E), 1)
    m1 = jnp.max(logits, axis=1, keepdims=True)
    e1 = jnp.min(jnp.where(logits == m1, ie, E), axis=1, keepdims=True)
    masked = jnp.where(ie == e1, -jnp.inf, logits)
    m2 = jnp.max(masked, axis=1, keepdims=True)
    e2 = jnp.min(jnp.where(masked == m2, ie, E), axis=1, keepdims=True)
    w1 = jax.nn.sigmoid(m1 - m2)
    w1_ref[...] = jnp.broadcast_to(w1, (T, 16))
    w2_ref[...] = jnp.broadcast_to(1.0 - w1, (T, 16))

    oh1 = (ie == e1).astype(jnp.float32)
    oh2 = (ie == e2).astype(jnp.float32)
    ohb = jnp.concatenate([oh1, oh2], axis=1).astype(jnp.bfloat16)
    it = lax.broadcasted_iota(jnp.int32, (T, T), 0)
    jt = lax.broadcasted_iota(jnp.int32, (T, T), 1)
    tri = (jt < it).astype(jnp.bfloat16)
    cb = jnp.dot(tri, ohb, preferred_element_type=jnp.float32)
    c1 = cb[:, :E]
    c2 = cb[:, E:]
    cnt1 = jnp.sum(oh1, axis=0, keepdims=True)
    cnt2 = jnp.sum(oh2, axis=0, keepdims=True)
    cnt = cnt1 + cnt2
    used = jnp.floor((cnt + (BT - 1)) * (1.0 / BT))

    iee = lax.broadcasted_iota(jnp.int32, (E, E), 0)
    jee = lax.broadcasted_iota(jnp.int32, (E, E), 1)
    upper = (iee < jee).astype(jnp.float32)
    used8 = jnp.broadcast_to(used, (E, E))
    start = jnp.dot(used8, upper, preferred_element_type=jnp.float32)[0:1]
    pad_off = start * BT

    pos1 = jnp.sum(oh1 * (pad_off + c1), axis=1, keepdims=True)
    pos2 = jnp.sum(oh2 * (pad_off + cnt1 + c2), axis=1, keepdims=True)
    pos1_ref[...] = pos1.astype(jnp.int32)
    pos2_ref[...] = pos2.astype(jnp.int32)

    ibf = lax.broadcasted_iota(jnp.int32, (NB, E), 0).astype(jnp.float32)
    ebf = lax.broadcasted_iota(jnp.int32, (NB, E), 1).astype(jnp.float32)
    startb = jnp.broadcast_to(start, (NB, E))
    usedb = jnp.broadcast_to(used, (NB, E))
    inr = jnp.logical_and(ibf >= startb, ibf < startb + usedb)
    bexp = jnp.sum(jnp.where(inr, ebf, 0.0), axis=1, keepdims=True)
    bval = jnp.sum(jnp.where(inr, 1.0, 0.0), axis=1, keepdims=True)
    ef = lax.broadcasted_iota(jnp.int32, (1, E), 1).astype(jnp.float32)
    laste = jnp.max(jnp.where(used > 0, ef, -1.0))
    bexp_ref[...] = jnp.where(bval > 0, bexp, laste).astype(jnp.int32)
    bval_ref[...] = (bval > 0).astype(jnp.int32)


_router = pl.pallas_call(
    _router_body,
    out_shape=(
        jax.ShapeDtypeStruct((T, 1), jnp.int32),
        jax.ShapeDtypeStruct((T, 1), jnp.int32),
        jax.ShapeDtypeStruct((T, 16), jnp.float32),
        jax.ShapeDtypeStruct((T, 16), jnp.float32),
        jax.ShapeDtypeStruct((NB, 1), jnp.int32),
        jax.ShapeDtypeStruct((NB, 1), jnp.int32),
    ),
)


@functools.partial(
    pl.kernel,
    out_type=jax.ShapeDtypeStruct((NPAD, H), jnp.float32),
    mesh=plsc.VectorSubcoreMesh(core_axis_name="c", subcore_axis_name="s",
                                num_cores=NC, num_subcores=NS),
    scratch_types=[
        pltpu.VMEM((TPW, H), jnp.float32),
        pltpu.VMEM((TPW,), jnp.int32),
        pltpu.VMEM((TPW,), jnp.int32),
        pltpu.SemaphoreType.DMA,
        pltpu.SemaphoreType.DMA,
    ],
)
def _dispatch(x_hbm, pos1_hbm, pos2_hbm, xs_hbm, xrows, p1v, p2v, s1, s2):
    wid = lax.axis_index("s") * NC + lax.axis_index("c")
    base = wid * TPW
    pltpu.sync_copy(x_hbm.at[pl.ds(base, TPW)], xrows)
    pltpu.sync_copy(pos1_hbm.at[pl.ds(base, TPW)], p1v)
    pltpu.sync_copy(pos2_hbm.at[pl.ds(base, TPW)], p2v)
    c1 = pltpu.async_copy(xrows, xs_hbm.at[p1v], s1)
    c2 = pltpu.async_copy(xrows, xs_hbm.at[p2v], s2)
    c1.wait()
    c2.wait()


def _ffn_body(bexp_ref, bval_ref, xs_ref, W1_ref, W3_ref, W2_ref, ys_ref):
    b = pl.program_id(0)

    @pl.when(bval_ref[b] != 0)
    def _():
        xb = xs_ref[...]
        acc = jnp.zeros((BT, H), jnp.float32)
        for fc in range(NFC):
            w1c = W1_ref[0, :, fc * FC:(fc + 1) * FC]
            w3c = W3_ref[0, :, fc * FC:(fc + 1) * FC]
            w2c = W2_ref[0, fc * FC:(fc + 1) * FC, :]
            h1 = jnp.dot(xb, w1c, preferred_element_type=jnp.float32)
            h3 = jnp.dot(xb, w3c, preferred_element_type=jnp.float32)
            act = h1 * jax.nn.sigmoid(h1) * h3
            acc = acc + jnp.dot(act, w2c, preferred_element_type=jnp.float32)
        ys_ref[...] = acc


_ffn = pl.pallas_call(
    _ffn_body,
    grid_spec=pltpu.PrefetchScalarGridSpec(
        num_scalar_prefetch=2,
        grid=(NB,),
        in_specs=[
            pl.BlockSpec((BT, H), lambda b, be, bv: (b, 0)),
            pl.BlockSpec((1, H, F), lambda b, be, bv: (be[b], 0, 0)),
            pl.BlockSpec((1, H, F), lambda b, be, bv: (be[b], 0, 0)),
            pl.BlockSpec((1, F, H), lambda b, be, bv: (be[b], 0, 0)),
        ],
        out_specs=pl.BlockSpec((BT, H), lambda b, be, bv: (b, 0)),
    ),
    out_shape=jax.ShapeDtypeStruct((NPAD, H), jnp.float32),
)


@functools.partial(
    pl.kernel,
    out_type=jax.ShapeDtypeStruct((T, H), jnp.float32),
    mesh=plsc.VectorSubcoreMesh(core_axis_name="c", subcore_axis_name="s",
                                num_cores=NC, num_subcores=NS),
    scratch_types=[
        pltpu.VMEM((TPW, H), jnp.float32),
        pltpu.VMEM((TPW, H), jnp.float32),
        pltpu.VMEM((TPW,), jnp.int32),
        pltpu.VMEM((TPW,), jnp.int32),
        pltpu.VMEM((TPW, 16), jnp.float32),
        pltpu.VMEM((TPW, 16), jnp.float32),
        pltpu.SemaphoreType.DMA,
        pltpu.SemaphoreType.DMA,
    ],
)
def _combine(ys_hbm, pos1_hbm, pos2_hbm, w1_hbm, w2_hbm, out_hbm,
             y1v, y2v, p1v, p2v, w1v, w2v, s1, s2):
    wid = lax.axis_index("s") * NC + lax.axis_index("c")
    base = wid * TPW
    pltpu.sync_copy(pos1_hbm.at[pl.ds(base, TPW)], p1v)
    pltpu.sync_copy(pos2_hbm.at[pl.ds(base, TPW)], p2v)
    pltpu.sync_copy(w1_hbm.at[pl.ds(base, TPW)], w1v)
    pltpu.sync_copy(w2_hbm.at[pl.ds(base, TPW)], w2v)
    c1 = pltpu.async_copy(ys_hbm.at[p1v], y1v, s1)
    c2 = pltpu.async_copy(ys_hbm.at[p2v], y2v, s2)
    c1.wait()
    c2.wait()

    def row(r, carry):
        wg1 = w1v[r, pl.ds(0, 16)]
        wg2 = w2v[r, pl.ds(0, 16)]
        for c0 in range(0, H, 16):
            y1v[r, pl.ds(c0, 16)] = (wg1 * y1v[r, pl.ds(c0, 16)]
                                     + wg2 * y2v[r, pl.ds(c0, 16)])
        return carry

    lax.fori_loop(0, TPW, row, 0)
    pltpu.sync_copy(y1v, out_hbm.at[pl.ds(base, TPW)])


def kernel(x, router_w, W1, W3, W2):
    b, l, h = x.shape
    x2 = x.reshape(T, H)
    pos1, pos2, w1, w2, bexp, bval = _router(x2, router_w)
    pos1 = pos1.reshape(T)
    pos2 = pos2.reshape(T)
    xs = _dispatch(x2, pos1, pos2)
    ys = _ffn(bexp.reshape(NB), bval.reshape(NB), xs, W1, W3, W2)
    out = _combine(ys, pos1, pos2, w1, w2)
    return out.reshape(b, l, h)
